# baseline (speedup 1.0000x reference)
import jax
import jax.numpy as jnp
from jax import lax
from jax.experimental import pallas as pl
from jax.experimental.pallas import tpu as pltpu
from jax.experimental.pallas import tpu_sc as plsc
from jax._src.pallas import mpmd as _mpmd

N_USERS = 50000
N_POIS = 50000
DIM = 128
NNZ = 600000
B = 16384

NC = 2
NS = 16
NW = NC * NS
CHUNK = 18752
LAST = NNZ - (NW - 1) * CHUNK
G = 128
NB = 148
CPAD = NB * G
SEG = 1024
NSEG = 20
RPAD = NSEG * SEG
MROWS = 392
R = 256
RB = 64
NBLK = R // RB
EXT = 16
LANE = 16


def _w_id():
    return lax.axis_index("c") * NS + lax.axis_index("s")


def _segsum_body(z_hbm, m_hbm, rows_hbm, cols_hbm, vals_hbm, mask_hbm, out_hbm,
                 rows_v, cols_v, vals_v, prev_v,
                 mrow0_v, mrow1_v, ring_v, rid_v,
                 msk0_v, msk1_v,
                 erow_v, ecol_v, eval_v, emrow_v,
                 sem0, sem1):
    del z_hbm
    w = _w_id()
    start = pl.multiple_of(w * CHUNK, 64)
    count = jnp.where(w == NW - 1, LAST, CHUNK)
    lane = lax.iota(jnp.int32, LANE)
    izero16 = jnp.zeros((LANE,), jnp.int32)

    for j in range((RPAD - LAST) // LANE):
        rows_v[pl.ds(LANE + LAST + j * LANE, LANE)] = izero16

    @pl.when(w < NW - 1)
    def _():
        pltpu.sync_copy(rows_hbm.at[pl.ds(start, CHUNK)], rows_v.at[pl.ds(LANE, CHUNK)])
        pltpu.sync_copy(cols_hbm.at[pl.ds(start, CHUNK)], cols_v.at[pl.ds(0, CHUNK)])
        pltpu.sync_copy(vals_hbm.at[pl.ds(start, CHUNK)], vals_v.at[pl.ds(0, CHUNK)])

    @pl.when(w == NW - 1)
    def _():
        pltpu.sync_copy(rows_hbm.at[pl.ds(start, LAST)], rows_v.at[pl.ds(LANE, LAST)])
        pltpu.sync_copy(cols_hbm.at[pl.ds(start, LAST)], cols_v.at[pl.ds(0, LAST)])
        pltpu.sync_copy(vals_hbm.at[pl.ds(start, LAST)], vals_v.at[pl.ds(0, LAST)])

    @pl.when(w > 0)
    def _():
        pltpu.sync_copy(rows_hbm.at[pl.ds(start - 8, 8)], prev_v.at[pl.ds(0, 8)])

    prev_row = jnp.where(w > 0, prev_v[pl.ds(0, LANE)][7], -1)
    rows_v[pl.ds(0, LANE)] = jnp.full((LANE,), prev_row, jnp.int32)
    ol_row = rows_v[pl.ds(count, LANE)][LANE - 1]

    def gather_desc(b, buf, sem):
        return pltpu.make_async_copy(
            m_hbm.at[cols_v.at[pl.ds(b * G, G)]], buf, sem)

    prow16 = jnp.full((LANE,), prev_row, jnp.int32)
    dimidx = [d * LANE + lane for d in range(DIM // LANE)]
    fzero16 = jnp.zeros((LANE,), jnp.float32)

    def mdesc(s, buf, sem):
        return pltpu.make_async_copy(
            mask_hbm.at[rows_v.at[pl.ds(LANE + s * SEG, SEG)]], buf, sem)

    def compact_seg(s, mbuf, n):
        def cg(g, n):
            e0 = s * SEG + g * LANE
            eof = jnp.minimum(e0, count - LANE)
            rva = rows_v[pl.ds(LANE + eof, LANE)]
            cva = cols_v[pl.ds(eof, LANE)]
            vva = vals_v[pl.ds(eof, LANE)]
            mk = mbuf[pl.ds(g * LANE, LANE)]
            keep = jnp.logical_and(mk != 0, e0 + lane < count)
            plsc.store_compressed(rows_v.at[pl.ds(LANE + n, LANE)], rva, mask=keep)
            plsc.store_compressed(cols_v.at[pl.ds(n, LANE)], cva, mask=keep)
            plsc.store_compressed(vals_v.at[pl.ds(n, LANE)], vva, mask=keep)
            pc = plsc.all_reduce_population_count(keep)
            return n + pc[0]
        return lax.fori_loop(0, SEG // LANE, cg, n)

    mdesc(0, msk0_v, sem0).start()

    def seg_pair(s2, n):
        s = 2 * s2
        mdesc(s + 1, msk1_v, sem1).start()
        mdesc(s, msk0_v, sem0).wait()
        n = compact_seg(s, msk0_v, n)

        @pl.when(s2 < NSEG // 2 - 1)
        def _():
            mdesc(s + 2, msk0_v, sem0).start()

        mdesc(s + 1, msk1_v, sem1).wait()
        n = compact_seg(s + 1, msk1_v, n)
        return n

    cnt = lax.fori_loop(0, NSEG // 2, seg_pair, jnp.int32(0))
    cl_row = rows_v[pl.ds(cnt, LANE)][LANE - 1]

    rows_v[pl.ds(LANE + cnt, LANE)] = jnp.full((LANE,), cl_row, jnp.int32)
    vals_v[pl.ds(cnt, LANE)] = fzero16

    def zc(j, _):
        cols_v[pl.ds(cnt + j * LANE, LANE)] = izero16
        return 0
    lax.fori_loop(0, 2 * G // LANE + 2, zc, 0)

    cnt2 = jnp.bitwise_and(cnt + LANE - 1, ~(LANE - 1))
    tc = (cnt2 + 2 * G - 1) // (2 * G)

    def flush_block(fb):
        fbm = fb & (NBLK - 1)
        base = pl.multiple_of(fbm * RB, RB)
        pltpu.sync_copy(ring_v.at[pl.ds(base, RB)], out_hbm.at[rid_v.at[fbm]])

    def consume_batch(b, buf, carry):
        lo = b * G
        hi = jnp.minimum(lo + G, cnt2)

        def group(g, c):
            rc = c[0]
            acc = list(c[1:])
            e0 = lo + g * LANE
            rva = rows_v[pl.ds(LANE + e0, LANE)]
            rvb = rows_v[pl.ds(LANE - 1 + e0, LANE)]
            vv = vals_v[pl.ds(e0, LANE)]
            proc = rva != prow16
            nrun = rva != rvb
            nrun_proc = jnp.logical_and(nrun, proc).astype(jnp.int32)
            incl = plsc.cumsum(nrun_proc)
            oid = rc + incl - 1
            zf_vec = jnp.where(jnp.logical_and(nrun, proc), 0.0, 1.0)
            wv_vec = jnp.where(proc, vv, 0.0)
            demit = jnp.logical_and(nrun, rvb != prow16).astype(jnp.int32)
            cm1 = oid - 1
            blkv = jnp.bitwise_and(jnp.right_shift(cm1, 6), NBLK - 1)
            slotv = jnp.bitwise_and(cm1, RB - 1)
            plsc.store_scatter(rid_v, [blkv, slotv], rvb, mask=demit != 0)
            sring = jnp.bitwise_and(cm1, R - 1)
            for i in range(LANE):
                pm = jnp.full((LANE,), demit[i] != 0)
                s16 = jnp.full((LANE,), sring[i], jnp.int32)
                zf = zf_vec[i]
                wv = wv_vec[i]
                e16 = jnp.full((LANE,), e0 + i - lo, jnp.int32)
                for d in range(DIM // LANE):
                    plsc.store_scatter(ring_v, [s16, dimidx[d]], acc[d],
                                       mask=pm)
                    md = plsc.load_gather(buf, [e16, dimidx[d]])
                    acc[d] = acc[d] * zf + wv * md
            return (rc + incl[LANE - 1],) + tuple(acc)

        ngroups = (hi - lo) // LANE
        rc, fb = carry[0], carry[1]
        inner = lax.fori_loop(0, ngroups, group, (rc,) + carry[2:])
        rc = inner[0]
        acc = inner[1:]

        def fcnd(st):
            return (st[1] + 1) * RB <= st[0] - 1

        def fbdy(st):
            flush_block(st[1])
            return (st[0], st[1] + 1)

        rc, fb = lax.while_loop(fcnd, fbdy, (rc, fb))
        return (rc, fb) + tuple(acc)

    @pl.when(tc > 0)
    def _():
        gather_desc(0, mrow0_v, sem0).start()

    def batch_pair(b2, carry):
        b = 2 * b2
        gather_desc(b + 1, mrow1_v, sem1).start()
        gather_desc(b, mrow0_v, sem0).wait()
        carry = consume_batch(b, mrow0_v, carry)

        @pl.when(b2 < tc - 1)
        def _():
            gather_desc(b + 2, mrow0_v, sem0).start()

        gather_desc(b + 1, mrow1_v, sem1).wait()
        carry = consume_batch(b + 1, mrow1_v, carry)
        return carry

    carry0 = (jnp.int32(0), jnp.int32(0)) + (fzero16,) * (DIM // LANE)
    carry = lax.fori_loop(0, tc, batch_pair, carry0)
    rc, fb = carry[0], carry[1]
    macc = carry[2:]
    has_runs = rc > 0

    def ext_cond(c):
        return jnp.logical_and(c[1], c[0] < NNZ)

    def ext_body(c):
        gpos = pl.multiple_of(c[0], EXT)
        acc = c[2:]
        pltpu.sync_copy(rows_hbm.at[pl.ds(gpos, EXT)], erow_v.at[pl.ds(0, EXT)])
        pltpu.sync_copy(cols_hbm.at[pl.ds(gpos, EXT)], ecol_v)
        pltpu.sync_copy(vals_hbm.at[pl.ds(gpos, EXT)], eval_v.at[pl.ds(0, EXT)])
        pltpu.async_copy(m_hbm.at[ecol_v], emrow_v, sem0).wait()

        def eb(e, c2):
            cont2 = c2[0]
            acc2 = c2[1:]
            m = jnp.logical_and(cont2, erow_v[pl.ds(e, LANE)][0] == ol_row)
            vv = jnp.where(m, eval_v[pl.ds(e, LANE)][0], 0.0)
            e16 = jnp.full((LANE,), e, jnp.int32)
            out = []
            for d in range(DIM // LANE):
                md = plsc.load_gather(emrow_v, [e16, dimidx[d]])
                out.append(acc2[d] + vv * md)
            return (m,) + tuple(out)

        inner = lax.fori_loop(0, EXT, eb, (c[1],) + acc)
        return (gpos + EXT, inner[0]) + inner[1:]

    do_ext = jnp.logical_and(has_runs, cl_row == ol_row)
    ext0 = (start + count, do_ext) + macc
    ext = lax.while_loop(ext_cond, ext_body, ext0)
    eacc = ext[2:]

    @pl.when(has_runs)
    def _():
        t = rc - 1
        s16 = jnp.full((LANE,), jnp.bitwise_and(t, R - 1), jnp.int32)
        for d in range(DIM // LANE):
            plsc.store_scatter(ring_v, [s16, dimidx[d]], eacc[d])
        tb16 = jnp.full((LANE,), jnp.bitwise_and(jnp.right_shift(t, 6), NBLK - 1),
                        jnp.int32)
        ts16 = jnp.full((LANE,), jnp.bitwise_and(t, RB - 1), jnp.int32)
        plsc.store_scatter(rid_v, [tb16, ts16],
                           jnp.full((LANE,), cl_row, jnp.int32),
                           mask=lane == 0)

    def gcnd(st):
        return (st + 1) * RB <= rc

    def gbdy(st):
        flush_block(st)
        return st + 1

    fb = lax.while_loop(gcnd, gbdy, fb)
    rem = rc - fb * RB

    @pl.when(rem > 0)
    def _():
        fbm = fb & (NBLK - 1)
        base = fbm * RB
        km1 = jnp.full((LANE,), base + rem - 1, jnp.int32)
        lid = plsc.load_gather(rid_v, [jnp.full((LANE,), fbm, jnp.int32),
                                       jnp.full((LANE,), rem - 1, jnp.int32)])
        lrow = [plsc.load_gather(ring_v, [km1, dimidx[d]])
                for d in range(DIM // LANE)]

        def pad(j, _):
            p = j >= rem
            pm = jnp.full((LANE,), p)
            j16 = jnp.full((LANE,), base + j, jnp.int32)
            for d in range(DIM // LANE):
                plsc.store_scatter(ring_v, [j16, dimidx[d]], lrow[d], mask=pm)
            plsc.store_scatter(rid_v, [jnp.full((LANE,), fbm, jnp.int32),
                                       jnp.full((LANE,), j, jnp.int32)],
                               lid, mask=jnp.logical_and(pm, lane == 0))
            return 0

        lax.fori_loop(0, RB, pad, 0)
        flush_block(fb)


def _gather_body(fm_hbm, selu_hbm, out_hbm, idx_v, rows_v, sem):
    w = _w_id()
    bpw = B // NW
    base = w * bpw
    pltpu.sync_copy(selu_hbm.at[pl.ds(base, bpw)], idx_v)
    pltpu.async_copy(fm_hbm.at[idx_v], rows_v, sem).wait()
    pltpu.sync_copy(rows_v, out_hbm.at[pl.ds(base, bpw)])


def _mask_body(mz_hbm, selu_hbm, out_hbm, idx_v, ones_v, sem):
    del mz_hbm
    w = _w_id()
    bpw = B // NW
    base = w * bpw
    for j in range(bpw // 128):
        pltpu.sync_copy(selu_hbm.at[pl.ds(base + j * 128, 128)], idx_v.at[j])
    for j in range(128 // LANE):
        ones_v[pl.ds(j * LANE, LANE)] = jnp.ones((LANE,), jnp.int32)
    for j in range(bpw // 128):
        pltpu.async_copy(ones_v, out_hbm.at[idx_v.at[j]], sem).wait()


_MESH = plsc.VectorSubcoreMesh(core_axis_name="c", subcore_axis_name="s")

_SEGSUM = _mpmd._mpmd_map(
    [(_MESH, _segsum_body)],
    jax.ShapeDtypeStruct((N_USERS, DIM), jnp.float32),
    input_output_aliases={0: 0},
    compiler_params=pltpu.CompilerParams(needs_layout_passes=False),
    scratch_types=[
        pltpu.VMEM((RPAD + 2 * LANE,), jnp.int32),
        pltpu.VMEM((CHUNK + 2 * G + 4 * LANE,), jnp.int32),
        pltpu.VMEM((CHUNK + 2 * LANE,), jnp.float32),
        pltpu.VMEM((LANE,), jnp.int32),
        pltpu.VMEM((G, DIM), jnp.float32),
        pltpu.VMEM((G, DIM), jnp.float32),
        pltpu.VMEM((R, DIM), jnp.float32),
        pltpu.VMEM((NBLK, RB), jnp.int32),
        pltpu.VMEM((SEG,), jnp.int32),
        pltpu.VMEM((SEG,), jnp.int32),
        pltpu.VMEM((EXT + LANE,), jnp.int32),
        pltpu.VMEM((EXT,), jnp.int32),
        pltpu.VMEM((EXT + LANE,), jnp.float32),
        pltpu.VMEM((EXT, DIM), jnp.float32),
        pltpu.SemaphoreType.DMA,
        pltpu.SemaphoreType.DMA,
    ],
)

_MASKSCAT = _mpmd._mpmd_map(
    [(_MESH, _mask_body)],
    jax.ShapeDtypeStruct((MROWS * DIM,), jnp.int32),
    input_output_aliases={0: 0},
    compiler_params=pltpu.CompilerParams(needs_layout_passes=False),
    scratch_types=[
        pltpu.VMEM((B // NW // 128, 128), jnp.int32),
        pltpu.VMEM((128,), jnp.int32),
        pltpu.SemaphoreType.DMA,
    ],
)

_GATHER = pl.kernel(
    _gather_body,
    out_type=jax.ShapeDtypeStruct((B, DIM), jnp.float32),
    mesh=plsc.VectorSubcoreMesh(core_axis_name="c", subcore_axis_name="s"),
    scratch_types=[
        pltpu.VMEM((B // NW,), jnp.int32),
        pltpu.VMEM((B // NW, DIM), jnp.float32),
        pltpu.SemaphoreType.DMA,
    ],
)

_MBLK = 1000


def _mm_zero_body(a_ref, w_ref, m_ref, z_ref, mz_ref):
    m_ref[...] = jnp.dot(a_ref[...], w_ref[...],
                         preferred_element_type=jnp.float32)
    z_ref[...] = jnp.zeros_like(z_ref)
    mz_ref[...] = jnp.zeros_like(mz_ref)


_EBLK = 512


def _epilogue_body(pm_ref, ue_ref, w_ref, b_ref, out_ref):
    um = jnp.dot(ue_ref[...], w_ref[...], preferred_element_type=jnp.float32)
    out_ref[...] = jnp.maximum(pm_ref[...] + um + b_ref[...], 0.0)


def kernel(all_out_going_embs, user_embs, selected_u, adj_rows, adj_cols, adj_vals, user_weight, poi_weight, bias):
    M, Z, MZ = pl.pallas_call(
        _mm_zero_body,
        grid=(N_POIS // _MBLK,),
        in_specs=[
            pl.BlockSpec((_MBLK, DIM), lambda i: (i, 0)),
            pl.BlockSpec((DIM, DIM), lambda i: (0, 0)),
        ],
        out_specs=[
            pl.BlockSpec((_MBLK, DIM), lambda i: (i, 0)),
            pl.BlockSpec((_MBLK, DIM), lambda i: (i, 0)),
            pl.BlockSpec((MROWS, DIM), lambda i: (0, 0)),
        ],
        out_shape=[
            jax.ShapeDtypeStruct((N_POIS, DIM), jnp.float32),
            jax.ShapeDtypeStruct((N_USERS, DIM), jnp.float32),
            jax.ShapeDtypeStruct((MROWS, DIM), jnp.int32),
        ],
    )(all_out_going_embs, poi_weight)

    mask = _MASKSCAT(MZ.reshape(MROWS * DIM), selected_u)
    full_msg = _SEGSUM(Z, M, adj_rows, adj_cols, adj_vals, mask)
    poi_message = _GATHER(full_msg, selected_u)

    out = pl.pallas_call(
        _epilogue_body,
        grid=(B // _EBLK,),
        in_specs=[
            pl.BlockSpec((_EBLK, DIM), lambda i: (i, 0)),
            pl.BlockSpec((_EBLK, DIM), lambda i: (i, 0)),
            pl.BlockSpec((DIM, DIM), lambda i: (0, 0)),
            pl.BlockSpec((1, DIM), lambda i: (0, 0)),
        ],
        out_specs=pl.BlockSpec((_EBLK, DIM), lambda i: (i, 0)),
        out_shape=jax.ShapeDtypeStruct((B, DIM), jnp.float32),
    )(poi_message, user_embs, user_weight, bias.reshape(1, DIM))
    return out


# speedup vs baseline: 1.7980x; 1.7980x over previous
import jax
import jax.numpy as jnp
from jax import lax
from jax.experimental import pallas as pl
from jax.experimental.pallas import tpu as pltpu
from jax.experimental.pallas import tpu_sc as plsc
from jax._src.pallas import mpmd as _mpmd

N_USERS = 50000
N_POIS = 50000
DIM = 128
NNZ = 600000
B = 16384

NC = 2
NS = 16
NW = NC * NS
CHUNK = 18752
LAST = NNZ - (NW - 1) * CHUNK
G = 128
NB = 148
CPAD = NB * G
BM = 1664
R = 256
RB = 64
NBLK = R // RB
EXT = 16
LANE = 16


def _w_id():
    return lax.axis_index("c") * NS + lax.axis_index("s")


def _segsum_body(z_hbm, m_hbm, rows_hbm, cols_hbm, vals_hbm, bmaps_hbm, out_hbm,
                 rows_v, cols_v, vals_v, prev_v,
                 mrow0_v, mrow1_v, ring_v, rid_v,
                 bma_v, bmt_v,
                 erow_v, ecol_v, eval_v, emrow_v,
                 sem0, sem1):
    del z_hbm
    w = _w_id()
    start = pl.multiple_of(w * CHUNK, 64)
    count = jnp.where(w == NW - 1, LAST, CHUNK)
    lane = lax.iota(jnp.int32, LANE)
    izero16 = jnp.zeros((LANE,), jnp.int32)

    @pl.when(w < NW - 1)
    def _():
        pltpu.sync_copy(rows_hbm.at[pl.ds(start, CHUNK)], rows_v.at[pl.ds(LANE, CHUNK)])
        pltpu.sync_copy(cols_hbm.at[pl.ds(start, CHUNK)], cols_v.at[pl.ds(0, CHUNK)])
        pltpu.sync_copy(vals_hbm.at[pl.ds(start, CHUNK)], vals_v.at[pl.ds(0, CHUNK)])

    @pl.when(w == NW - 1)
    def _():
        pltpu.sync_copy(rows_hbm.at[pl.ds(start, LAST)], rows_v.at[pl.ds(LANE, LAST)])
        pltpu.sync_copy(cols_hbm.at[pl.ds(start, LAST)], cols_v.at[pl.ds(0, LAST)])
        pltpu.sync_copy(vals_hbm.at[pl.ds(start, LAST)], vals_v.at[pl.ds(0, LAST)])

    @pl.when(w > 0)
    def _():
        pltpu.sync_copy(rows_hbm.at[pl.ds(start - 8, 8)], prev_v.at[pl.ds(0, 8)])

    prev_row = jnp.where(w > 0, prev_v[pl.ds(0, LANE)][7], -1)
    rows_v[pl.ds(0, LANE)] = jnp.full((LANE,), prev_row, jnp.int32)
    ol_row = rows_v[pl.ds(count, LANE)][LANE - 1]

    def gather_desc(b, buf, sem):
        return pltpu.make_async_copy(
            m_hbm.at[cols_v.at[pl.ds(b * G, G)]], buf, sem)

    prow16 = jnp.full((LANE,), prev_row, jnp.int32)
    dimidx = [d * LANE + lane for d in range(DIM // LANE)]
    fzero16 = jnp.zeros((LANE,), jnp.float32)

    pltpu.sync_copy(bmaps_hbm.at[0], bma_v.at[pl.ds(0, BM)])

    def orone(t, _):
        pltpu.sync_copy(bmaps_hbm.at[t], bmt_v.at[pl.ds(0, BM)])

        def orj(j, _):
            a = bma_v[pl.ds(j * LANE, LANE)]
            b = bmt_v[pl.ds(j * LANE, LANE)]
            bma_v[pl.ds(j * LANE, LANE)] = jnp.bitwise_or(a, b)
            return 0

        lax.fori_loop(0, BM // LANE, orj, 0)
        return 0

    lax.fori_loop(1, NW, orone, 0)

    def cg(g, n):
        e0 = g * LANE
        rva = rows_v[pl.ds(LANE + e0, LANE)]
        cva = cols_v[pl.ds(e0, LANE)]
        vva = vals_v[pl.ds(e0, LANE)]
        word = plsc.load_gather(bma_v, [jnp.right_shift(rva, 5)])
        keep = jnp.bitwise_and(
            jnp.right_shift(word, jnp.bitwise_and(rva, 31)), 1) != 0
        plsc.store_compressed(rows_v.at[pl.ds(LANE + n, LANE)], rva, mask=keep)
        plsc.store_compressed(cols_v.at[pl.ds(n, LANE)], cva, mask=keep)
        plsc.store_compressed(vals_v.at[pl.ds(n, LANE)], vva, mask=keep)
        pc = plsc.all_reduce_population_count(keep)
        return n + pc[0]

    cnt = lax.fori_loop(0, count // LANE, cg, jnp.int32(0))
    cl_row = rows_v[pl.ds(cnt, LANE)][LANE - 1]

    rows_v[pl.ds(LANE + cnt, LANE)] = jnp.full((LANE,), cl_row, jnp.int32)
    vals_v[pl.ds(cnt, LANE)] = fzero16

    def zc(j, _):
        cols_v[pl.ds(cnt + j * LANE, LANE)] = izero16
        return 0
    lax.fori_loop(0, 2 * G // LANE + 2, zc, 0)

    cnt2 = jnp.bitwise_and(cnt + LANE - 1, ~(LANE - 1))
    tc = (cnt2 + 2 * G - 1) // (2 * G)

    def flush_block(fb):
        fbm = fb & (NBLK - 1)
        base = pl.multiple_of(fbm * RB, RB)
        pltpu.sync_copy(ring_v.at[pl.ds(base, RB)], out_hbm.at[rid_v.at[fbm]])

    def consume_batch(b, buf, carry):
        lo = b * G
        hi = jnp.minimum(lo + G, cnt2)

        def group(g, c):
            rc = c[0]
            acc = list(c[1:])
            e0 = lo + g * LANE
            rva = rows_v[pl.ds(LANE + e0, LANE)]
            rvb = rows_v[pl.ds(LANE - 1 + e0, LANE)]
            vv = vals_v[pl.ds(e0, LANE)]
            proc = rva != prow16
            nrun = rva != rvb
            nrun_proc = jnp.logical_and(nrun, proc).astype(jnp.int32)
            incl = plsc.cumsum(nrun_proc)
            oid = rc + incl - 1
            zf_vec = jnp.where(jnp.logical_and(nrun, proc), 0.0, 1.0)
            wv_vec = jnp.where(proc, vv, 0.0)
            demit = jnp.logical_and(nrun, rvb != prow16).astype(jnp.int32)
            cm1 = oid - 1
            blkv = jnp.bitwise_and(jnp.right_shift(cm1, 6), NBLK - 1)
            slotv = jnp.bitwise_and(cm1, RB - 1)
            plsc.store_scatter(rid_v, [blkv, slotv], rvb, mask=demit != 0)
            sring = jnp.bitwise_and(cm1, R - 1)
            for i in range(LANE):
                pm = jnp.full((LANE,), demit[i] != 0)
                s16 = jnp.full((LANE,), sring[i], jnp.int32)
                zf = zf_vec[i]
                wv = wv_vec[i]
                e16 = jnp.full((LANE,), e0 + i - lo, jnp.int32)
                for d in range(DIM // LANE):
                    plsc.store_scatter(ring_v, [s16, dimidx[d]], acc[d],
                                       mask=pm)
                    md = plsc.load_gather(buf, [e16, dimidx[d]])
                    acc[d] = acc[d] * zf + wv * md
            return (rc + incl[LANE - 1],) + tuple(acc)

        ngroups = (hi - lo) // LANE
        rc, fb = carry[0], carry[1]
        inner = lax.fori_loop(0, ngroups, group, (rc,) + carry[2:])
        rc = inner[0]
        acc = inner[1:]

        def fcnd(st):
            return (st[1] + 1) * RB <= st[0] - 1

        def fbdy(st):
            flush_block(st[1])
            return (st[0], st[1] + 1)

        rc, fb = lax.while_loop(fcnd, fbdy, (rc, fb))
        return (rc, fb) + tuple(acc)

    @pl.when(tc > 0)
    def _():
        gather_desc(0, mrow0_v, sem0).start()

    def batch_pair(b2, carry):
        b = 2 * b2
        gather_desc(b + 1, mrow1_v, sem1).start()
        gather_desc(b, mrow0_v, sem0).wait()
        carry = consume_batch(b, mrow0_v, carry)

        @pl.when(b2 < tc - 1)
        def _():
            gather_desc(b + 2, mrow0_v, sem0).start()

        gather_desc(b + 1, mrow1_v, sem1).wait()
        carry = consume_batch(b + 1, mrow1_v, carry)
        return carry

    carry0 = (jnp.int32(0), jnp.int32(0)) + (fzero16,) * (DIM // LANE)
    carry = lax.fori_loop(0, tc, batch_pair, carry0)
    rc, fb = carry[0], carry[1]
    macc = carry[2:]
    has_runs = rc > 0

    def ext_cond(c):
        return jnp.logical_and(c[1], c[0] < NNZ)

    def ext_body(c):
        gpos = pl.multiple_of(c[0], EXT)
        acc = c[2:]
        pltpu.sync_copy(rows_hbm.at[pl.ds(gpos, EXT)], erow_v.at[pl.ds(0, EXT)])
        pltpu.sync_copy(cols_hbm.at[pl.ds(gpos, EXT)], ecol_v)
        pltpu.sync_copy(vals_hbm.at[pl.ds(gpos, EXT)], eval_v.at[pl.ds(0, EXT)])
        pltpu.async_copy(m_hbm.at[ecol_v], emrow_v, sem0).wait()

        def eb(e, c2):
            cont2 = c2[0]
            acc2 = c2[1:]
            m = jnp.logical_and(cont2, erow_v[pl.ds(e, LANE)][0] == ol_row)
            vv = jnp.where(m, eval_v[pl.ds(e, LANE)][0], 0.0)
            e16 = jnp.full((LANE,), e, jnp.int32)
            out = []
            for d in range(DIM // LANE):
                md = plsc.load_gather(emrow_v, [e16, dimidx[d]])
                out.append(acc2[d] + vv * md)
            return (m,) + tuple(out)

        inner = lax.fori_loop(0, EXT, eb, (c[1],) + acc)
        return (gpos + EXT, inner[0]) + inner[1:]

    do_ext = jnp.logical_and(has_runs, cl_row == ol_row)
    ext0 = (start + count, do_ext) + macc
    ext = lax.while_loop(ext_cond, ext_body, ext0)
    eacc = ext[2:]

    @pl.when(has_runs)
    def _():
        t = rc - 1
        s16 = jnp.full((LANE,), jnp.bitwise_and(t, R - 1), jnp.int32)
        for d in range(DIM // LANE):
            plsc.store_scatter(ring_v, [s16, dimidx[d]], eacc[d])
        tb16 = jnp.full((LANE,), jnp.bitwise_and(jnp.right_shift(t, 6), NBLK - 1),
                        jnp.int32)
        ts16 = jnp.full((LANE,), jnp.bitwise_and(t, RB - 1), jnp.int32)
        plsc.store_scatter(rid_v, [tb16, ts16],
                           jnp.full((LANE,), cl_row, jnp.int32),
                           mask=lane == 0)

    def gcnd(st):
        return (st + 1) * RB <= rc

    def gbdy(st):
        flush_block(st)
        return st + 1

    fb = lax.while_loop(gcnd, gbdy, fb)
    rem = rc - fb * RB

    @pl.when(rem > 0)
    def _():
        fbm = fb & (NBLK - 1)
        base = fbm * RB
        km1 = jnp.full((LANE,), base + rem - 1, jnp.int32)
        lid = plsc.load_gather(rid_v, [jnp.full((LANE,), fbm, jnp.int32),
                                       jnp.full((LANE,), rem - 1, jnp.int32)])
        lrow = [plsc.load_gather(ring_v, [km1, dimidx[d]])
                for d in range(DIM // LANE)]

        def pad(j, _):
            p = j >= rem
            pm = jnp.full((LANE,), p)
            j16 = jnp.full((LANE,), base + j, jnp.int32)
            for d in range(DIM // LANE):
                plsc.store_scatter(ring_v, [j16, dimidx[d]], lrow[d], mask=pm)
            plsc.store_scatter(rid_v, [jnp.full((LANE,), fbm, jnp.int32),
                                       jnp.full((LANE,), j, jnp.int32)],
                               lid, mask=jnp.logical_and(pm, lane == 0))
            return 0

        lax.fori_loop(0, RB, pad, 0)
        flush_block(fb)


def _gather_body(fm_hbm, selu_hbm, out_hbm, idx_v, rows_v, sem):
    w = _w_id()
    bpw = B // NW
    base = w * bpw
    pltpu.sync_copy(selu_hbm.at[pl.ds(base, bpw)], idx_v)
    pltpu.async_copy(fm_hbm.at[idx_v], rows_v, sem).wait()
    pltpu.sync_copy(rows_v, out_hbm.at[pl.ds(base, bpw)])


def _mask_body(selu_hbm, out_hbm, idx_v, bm_v, sem):
    del sem
    w = _w_id()
    bpw = B // NW
    base = w * bpw
    lane = lax.iota(jnp.int32, LANE)
    pltpu.sync_copy(selu_hbm.at[pl.ds(base, bpw)], idx_v)

    def zb(j, _):
        bm_v[pl.ds(j * LANE, LANE)] = jnp.zeros((LANE,), jnp.int32)
        return 0

    lax.fori_loop(0, BM // LANE, zb, 0)

    def setbit(i, _):
        uid = idx_v[pl.ds(i, LANE)][0]
        wi = jnp.right_shift(uid, 5)
        old = bm_v[pl.ds(wi, LANE)][0]
        new = jnp.bitwise_or(old, jnp.left_shift(1, jnp.bitwise_and(uid, 31)))
        plsc.store_scatter(bm_v, [jnp.full((LANE,), wi, jnp.int32)],
                           jnp.full((LANE,), new, jnp.int32), mask=lane == 0)
        return 0

    lax.fori_loop(0, bpw, setbit, 0)
    pltpu.sync_copy(bm_v.at[pl.ds(0, BM)], out_hbm.at[w])


_MESH = plsc.VectorSubcoreMesh(core_axis_name="c", subcore_axis_name="s")

_SEGSUM = _mpmd._mpmd_map(
    [(_MESH, _segsum_body)],
    jax.ShapeDtypeStruct((N_USERS, DIM), jnp.float32),
    input_output_aliases={0: 0},
    compiler_params=pltpu.CompilerParams(needs_layout_passes=False),
    scratch_types=[
        pltpu.VMEM((CHUNK + 4 * LANE,), jnp.int32),
        pltpu.VMEM((CHUNK + 2 * G + 4 * LANE,), jnp.int32),
        pltpu.VMEM((CHUNK + 2 * LANE,), jnp.float32),
        pltpu.VMEM((LANE,), jnp.int32),
        pltpu.VMEM((G, DIM), jnp.float32),
        pltpu.VMEM((G, DIM), jnp.float32),
        pltpu.VMEM((R, DIM), jnp.float32),
        pltpu.VMEM((NBLK, RB), jnp.int32),
        pltpu.VMEM((BM + LANE,), jnp.int32),
        pltpu.VMEM((BM + LANE,), jnp.int32),
        pltpu.VMEM((EXT + LANE,), jnp.int32),
        pltpu.VMEM((EXT,), jnp.int32),
        pltpu.VMEM((EXT + LANE,), jnp.float32),
        pltpu.VMEM((EXT, DIM), jnp.float32),
        pltpu.SemaphoreType.DMA,
        pltpu.SemaphoreType.DMA,
    ],
)

_MASKSCAT = _mpmd._mpmd_map(
    [(_MESH, _mask_body)],
    jax.ShapeDtypeStruct((NW, BM), jnp.int32),
    compiler_params=pltpu.CompilerParams(needs_layout_passes=False),
    scratch_types=[
        pltpu.VMEM((B // NW,), jnp.int32),
        pltpu.VMEM((BM + LANE,), jnp.int32),
        pltpu.SemaphoreType.DMA,
    ],
)

_GATHER = pl.kernel(
    _gather_body,
    out_type=jax.ShapeDtypeStruct((B, DIM), jnp.float32),
    mesh=plsc.VectorSubcoreMesh(core_axis_name="c", subcore_axis_name="s"),
    scratch_types=[
        pltpu.VMEM((B // NW,), jnp.int32),
        pltpu.VMEM((B // NW, DIM), jnp.float32),
        pltpu.SemaphoreType.DMA,
    ],
)

_MBLK = 1000


def _mm_zero_body(a_ref, w_ref, m_ref, z_ref):
    m_ref[...] = jnp.dot(a_ref[...], w_ref[...],
                         preferred_element_type=jnp.float32)
    z_ref[...] = jnp.zeros_like(z_ref)


_EBLK = 512


def _epilogue_body(pm_ref, ue_ref, w_ref, b_ref, out_ref):
    um = jnp.dot(ue_ref[...], w_ref[...], preferred_element_type=jnp.float32)
    out_ref[...] = jnp.maximum(pm_ref[...] + um + b_ref[...], 0.0)


def kernel(all_out_going_embs, user_embs, selected_u, adj_rows, adj_cols, adj_vals, user_weight, poi_weight, bias):
    M, Z = pl.pallas_call(
        _mm_zero_body,
        grid=(N_POIS // _MBLK,),
        in_specs=[
            pl.BlockSpec((_MBLK, DIM), lambda i: (i, 0)),
            pl.BlockSpec((DIM, DIM), lambda i: (0, 0)),
        ],
        out_specs=[
            pl.BlockSpec((_MBLK, DIM), lambda i: (i, 0)),
            pl.BlockSpec((_MBLK, DIM), lambda i: (i, 0)),
        ],
        out_shape=[
            jax.ShapeDtypeStruct((N_POIS, DIM), jnp.float32),
            jax.ShapeDtypeStruct((N_USERS, DIM), jnp.float32),
        ],
    )(all_out_going_embs, poi_weight)

    bmaps = _MASKSCAT(selected_u)
    full_msg = _SEGSUM(Z, M, adj_rows, adj_cols, adj_vals, bmaps)
    poi_message = _GATHER(full_msg, selected_u)

    out = pl.pallas_call(
        _epilogue_body,
        grid=(B // _EBLK,),
        in_specs=[
            pl.BlockSpec((_EBLK, DIM), lambda i: (i, 0)),
            pl.BlockSpec((_EBLK, DIM), lambda i: (i, 0)),
            pl.BlockSpec((DIM, DIM), lambda i: (0, 0)),
            pl.BlockSpec((1, DIM), lambda i: (0, 0)),
        ],
        out_specs=pl.BlockSpec((_EBLK, DIM), lambda i: (i, 0)),
        out_shape=jax.ShapeDtypeStruct((B, DIM), jnp.float32),
    )(poi_message, user_embs, user_weight, bias.reshape(1, DIM))
    return out
